# Initial kernel scaffold; baseline (speedup 1.0000x reference)
import jax
import jax.numpy as jnp
from jax.experimental import pallas as pl


def kernel(x, edge_index, W1, b1, W2, b2):
    raise NotImplementedError("write your pallas kernel here")


# speedup vs baseline: 12.6150x; 12.6150x over previous
import functools

import jax
import jax.numpy as jnp
from jax import lax
from jax.experimental import pallas as pl
from jax.experimental.pallas import tpu as pltpu
from jax.experimental.pallas import tpu_sc as plsc

N = 10000
E = 320000
D = 128

NC = 2
NS = 16
NW = NC * NS
CHUNK = 128
CPT = 79
EPAD = NW * CPT * CHUNK
NPAD = 10240
RPT = NPAD // NS
HW = 16


def _deg_body(dst_hbm, out_hbm, dst_v, ones_v, acc_sh):
    cid = lax.axis_index("c")
    sid = lax.axis_index("s")
    wid = cid * NS + sid
    pltpu.sync_copy(dst_hbm.at[wid], dst_v)

    def fill(i, val):
        def body(_i, _):
            ones_v[_i, :] = jnp.full((HW,), val, jnp.float32)
            return 0
        lax.fori_loop(0, CHUNK, body, 0)

    fill(0, 0.0)
    base = sid * RPT
    for k in range(RPT // CHUNK):
        pltpu.sync_copy(ones_v, acc_sh.at[pl.ds(base + k * CHUNK, CHUNK)])
    plsc.subcore_barrier()

    fill(0, 1.0)

    def body(j, _):
        pltpu.sync_copy(ones_v, acc_sh.at[dst_v.at[j]], add=True)
        return 0
    lax.fori_loop(0, CPT, body, 0)
    plsc.subcore_barrier()
    pltpu.sync_copy(acc_sh.at[pl.ds(base, RPT)],
                    out_hbm.at[cid, pl.ds(base, RPT)])


def _agg_body(hp_hbm, src_hbm, dst_hbm, out_hbm, src_v, dst_v, rows_v, acc_sh):
    cid = lax.axis_index("c")
    sid = lax.axis_index("s")
    wid = cid * NS + sid
    pltpu.sync_copy(src_hbm.at[wid], src_v)
    pltpu.sync_copy(dst_hbm.at[wid], dst_v)

    def zrow(i, _):
        for j in range(D // 16):
            rows_v[i, pl.ds(j * 16, 16)] = jnp.zeros((16,), jnp.float32)
        return 0
    lax.fori_loop(0, CHUNK, zrow, 0)
    base = sid * RPT
    for k in range(RPT // CHUNK):
        pltpu.sync_copy(rows_v, acc_sh.at[pl.ds(base + k * CHUNK, CHUNK)])
    plsc.subcore_barrier()

    def body(j, _):
        pltpu.sync_copy(hp_hbm.at[src_v.at[j]], rows_v)
        pltpu.sync_copy(rows_v, acc_sh.at[dst_v.at[j]], add=True)
        return 0
    lax.fori_loop(0, CPT, body, 0)
    plsc.subcore_barrier()
    pltpu.sync_copy(acc_sh.at[pl.ds(base, RPT)],
                    out_hbm.at[cid, pl.ds(base, RPT)])


@functools.cache
def _sc_kernels():
    mesh = plsc.VectorSubcoreMesh(core_axis_name="c", subcore_axis_name="s",
                                  num_cores=NC, num_subcores=NS)
    deg = pl.kernel(
        _deg_body,
        out_type=jax.ShapeDtypeStruct((NC, NPAD, HW), jnp.float32),
        mesh=mesh,
        scratch_types=[
            pltpu.VMEM((CPT, CHUNK), jnp.int32),
            pltpu.VMEM((CHUNK, HW), jnp.float32),
            pltpu.VMEM_SHARED((NPAD, HW), jnp.float32),
        ],
    )
    agg = pl.kernel(
        _agg_body,
        out_type=jax.ShapeDtypeStruct((NC, NPAD, D), jnp.float32),
        mesh=mesh,
        scratch_types=[
            pltpu.VMEM((CPT, CHUNK), jnp.int32),
            pltpu.VMEM((CPT, CHUNK), jnp.int32),
            pltpu.VMEM((CHUNK, D), jnp.float32),
            pltpu.VMEM_SHARED((NPAD, D), jnp.float32),
        ],
    )
    return deg, agg


_BLK = 256
_GRID = NPAD // _BLK


def _dis(h0_ref, h1_ref):
    deg = h0_ref[:, 0:1] + h1_ref[:, 0:1] + 1.0
    return lax.rsqrt(deg)


def _mm_scale_body(h0_ref, h1_ref, x_ref, w_ref, o_ref):
    h = jnp.dot(x_ref[...], w_ref[...], preferred_element_type=jnp.float32)
    o_ref[...] = h * _dis(h0_ref, h1_ref)


def _mid_body(h0_ref, h1_ref, p0_ref, p1_ref, hp_ref, b_ref, w_ref, o_ref):
    dis = _dis(h0_ref, h1_ref)
    z = (p0_ref[...] + p1_ref[...] + hp_ref[...]) * dis + b_ref[...]
    h = jnp.maximum(z, 0.0)
    o_ref[...] = jnp.dot(h, w_ref[...],
                         preferred_element_type=jnp.float32) * dis


def _final_body(h0_ref, h1_ref, p0_ref, p1_ref, hp_ref, b_ref, o_ref):
    dis = _dis(h0_ref, h1_ref)
    z = (p0_ref[...] + p1_ref[...] + hp_ref[...]) * dis + b_ref[...]
    m = jnp.max(z, axis=1, keepdims=True)
    shifted = z - m
    lse = jnp.log(jnp.sum(jnp.exp(shifted), axis=1, keepdims=True))
    o_ref[...] = shifted - lse


def _row_spec(w):
    return pl.BlockSpec((_BLK, w), lambda i: (i, 0))


def _full_spec(r, c):
    return pl.BlockSpec((r, c), lambda i: (0, 0))


_OUT_SPEC = pl.BlockSpec((_BLK, D), lambda i: (i, 0))
_OUT_SHAPE = jax.ShapeDtypeStruct((NPAD, D), jnp.float32)

_tc1 = pl.pallas_call(
    _mm_scale_body,
    grid=(_GRID,),
    in_specs=[_row_spec(HW), _row_spec(HW), _row_spec(D), _full_spec(D, D)],
    out_specs=_OUT_SPEC,
    out_shape=_OUT_SHAPE,
)

_tc2 = pl.pallas_call(
    _mid_body,
    grid=(_GRID,),
    in_specs=[_row_spec(HW), _row_spec(HW), _row_spec(D), _row_spec(D),
              _row_spec(D), _full_spec(1, D), _full_spec(D, D)],
    out_specs=_OUT_SPEC,
    out_shape=_OUT_SHAPE,
)

_tc3 = pl.pallas_call(
    _final_body,
    grid=(_GRID,),
    in_specs=[_row_spec(HW), _row_spec(HW), _row_spec(D), _row_spec(D),
              _row_spec(D), _full_spec(1, D)],
    out_specs=_OUT_SPEC,
    out_shape=_OUT_SHAPE,
)


def kernel(x, edge_index, W1, b1, W2, b2):
    src = edge_index[0]
    dst = edge_index[1]
    pad = EPAD - E
    src_p = jnp.concatenate(
        [src, jnp.zeros((pad,), jnp.int32)]).reshape(NW, CPT, CHUNK)
    dst_p = jnp.concatenate(
        [dst, jnp.full((pad,), NPAD - 1, jnp.int32)]).reshape(NW, CPT, CHUNK)
    xp = jnp.concatenate([x, jnp.zeros((NPAD - N, D), x.dtype)])

    _deg, _agg = _sc_kernels()
    hist = _deg(dst_p)
    h0, h1 = hist[0], hist[1]
    hp1 = _tc1(h0, h1, xp, W1)
    p = _agg(hp1, src_p, dst_p)
    hp2 = _tc2(h0, h1, p[0], p[1], hp1, b1.reshape(1, D), W2)
    p2 = _agg(hp2, src_p, dst_p)
    outp = _tc3(h0, h1, p2[0], p2[1], hp2, b2.reshape(1, D))
    return outp[:N]
